# baseline (speedup 1.0000x reference)
import functools

import jax
import jax.numpy as jnp
from jax import lax
from jax.experimental import pallas as pl
from jax.experimental.pallas import tpu as pltpu
from jax.experimental.pallas import tpu_sc as plsc

_NC = 2
_NS = 16
_NW = _NC * _NS
_CH = 128
_NB = 3
_DEGW = 8


def _npad(n):
    return ((n + _NS + _NS - 1) // _NS) * _NS


def _run_pass(c, table, ridx_v, cidx_v, bufs, acc, sem_g, sem_s, zrows,
              nch0, nch1):
    @pl.when(c == 0)
    def _():
        def pair(g2, carry):
            bbase = g2 * 2 * _NB
            for p in range(2):
                @pl.when(g2 > 0)
                def _():
                    for i in range(_NB):
                        pltpu.make_async_copy(zrows, bufs[p][i],
                                              sem_s[p]).wait()
                for i in range(_NB):
                    j = bbase + p * _NB + i
                    pltpu.async_copy(table.at[ridx_v.at[j]], bufs[p][i],
                                     sem_g[p])
                for i in range(_NB):
                    j = bbase + p * _NB + i
                    pltpu.make_async_copy(zrows, bufs[p][i], sem_g[p]).wait()
                    pltpu.async_copy(bufs[p][i], acc.at[cidx_v.at[j]],
                                     sem_s[p], add=True)
            return carry

        lax.fori_loop(0, nch0 // (2 * _NB), pair, 0)
        if nch0:
            for p in range(2):
                for i in range(_NB):
                    pltpu.make_async_copy(zrows, bufs[p][i], sem_s[p]).wait()

    @pl.when(c == 1)
    def _():
        if nch1:
            ba, bb = bufs[0][0], bufs[0][1]
            ng = nch1 // 2
            pltpu.async_copy(table.at[ridx_v.at[0]], ba, sem_g[1])

            def sch2(g, carry):
                j = g * 2
                pltpu.make_async_copy(zrows, ba, sem_g[1]).wait()
                pltpu.async_copy(table.at[ridx_v.at[j + 1]], bb, sem_g[1])
                pltpu.sync_copy(ba, acc.at[cidx_v.at[j]], add=True)
                pltpu.make_async_copy(zrows, bb, sem_g[1]).wait()

                @pl.when(g + 1 < ng)
                def _():
                    pltpu.async_copy(table.at[ridx_v.at[j + 2]], ba,
                                     sem_g[1])
                pltpu.sync_copy(bb, acc.at[cidx_v.at[j + 1]], add=True)
                return carry

            lax.fori_loop(0, ng, sch2, 0)


@functools.lru_cache(maxsize=None)
def _make_q(n, d, nch0, nch1):
    npad = _npad(n)
    orow = npad // _NS
    zfull, zrem = orow // _CH, orow % _CH
    nchv = max(nch0, nch1)
    mesh = plsc.VectorSubcoreMesh(core_axis_name="c", subcore_axis_name="s",
                                  num_cores=_NC, num_subcores=_NS)

    @functools.partial(
        pl.kernel,
        out_type=jax.ShapeDtypeStruct((_NC, npad, d), jnp.float32),
        mesh=mesh,
        compiler_params=pltpu.CompilerParams(use_tc_tiling_on_sc=False),
        scratch_types=[
            pltpu.VMEM((nchv, _CH), jnp.int32),
            pltpu.VMEM((nchv, _CH), jnp.int32),
            [[pltpu.VMEM((_CH, d), jnp.float32) for _ in range(_NB)]
             for _ in range(2)],
            pltpu.VMEM_SHARED((npad, d), jnp.float32),
            [pltpu.SemaphoreType.DMA for _ in range(2)],
            [pltpu.SemaphoreType.DMA for _ in range(2)],
        ],
    )
    def qk(table, ridx, cidx, zrows, out, ridx_v, cidx_v, bufs, acc,
           sem_g, sem_s):
        c = lax.axis_index("c")
        s = lax.axis_index("s")
        base = s * orow
        for j in range(zfull):
            pltpu.sync_copy(zrows, acc.at[pl.ds(base + j * _CH, _CH)])
        if zrem:
            pltpu.sync_copy(zrows.at[pl.ds(0, zrem)],
                            acc.at[pl.ds(base + zfull * _CH, zrem)])

        @pl.when(c == 0)
        def _():
            if nch0:
                pltpu.sync_copy(ridx.at[s, pl.ds(0, nch0)],
                                ridx_v.at[pl.ds(0, nch0)])
                pltpu.sync_copy(cidx.at[s, pl.ds(0, nch0)],
                                cidx_v.at[pl.ds(0, nch0)])

        @pl.when(c == 1)
        def _():
            if nch1:
                pltpu.sync_copy(ridx.at[s, pl.ds(nch0, nch1)],
                                ridx_v.at[pl.ds(0, nch1)])
                pltpu.sync_copy(cidx.at[s, pl.ds(nch0, nch1)],
                                cidx_v.at[pl.ds(0, nch1)])

        plsc.subcore_barrier()
        _run_pass(c, table, ridx_v, cidx_v, bufs, acc, sem_g, sem_s, zrows,
                  nch0, nch1)
        plsc.subcore_barrier()
        pltpu.sync_copy(acc.at[pl.ds(s * orow, orow)],
                        out.at[c, pl.ds(s * orow, orow)])

    return qk


@functools.lru_cache(maxsize=None)
def _make_q2(n, d, nch0, nch1):
    npad = _npad(n)
    orow = npad // _NS
    zfull, zrem = orow // _CH, orow % _CH
    nchv = max(nch0, nch1)
    mesh = plsc.VectorSubcoreMesh(core_axis_name="c", subcore_axis_name="s",
                                  num_cores=_NC, num_subcores=_NS)

    @functools.partial(
        pl.kernel,
        out_type=jax.ShapeDtypeStruct((2, _NC, npad, d), jnp.float32),
        mesh=mesh,
        compiler_params=pltpu.CompilerParams(use_tc_tiling_on_sc=False),
        scratch_types=[
            pltpu.VMEM((nchv, _CH), jnp.int32),
            pltpu.VMEM((nchv, _CH), jnp.int32),
            [[pltpu.VMEM((_CH, d), jnp.float32) for _ in range(_NB)]
             for _ in range(2)],
            pltpu.VMEM_SHARED((npad, d), jnp.float32),
            [pltpu.SemaphoreType.DMA for _ in range(2)],
            [pltpu.SemaphoreType.DMA for _ in range(2)],
        ],
    )
    def qk(table_a, table_b, ridx, cidx, zrows, out, ridx_v, cidx_v, bufs,
           acc, sem_g, sem_s):
        c = lax.axis_index("c")
        s = lax.axis_index("s")
        base = s * orow

        def zero_own_slice():
            for j in range(zfull):
                pltpu.sync_copy(zrows, acc.at[pl.ds(base + j * _CH, _CH)])
            if zrem:
                pltpu.sync_copy(zrows.at[pl.ds(0, zrem)],
                                acc.at[pl.ds(base + zfull * _CH, zrem)])

        zero_own_slice()

        @pl.when(c == 0)
        def _():
            if nch0:
                pltpu.sync_copy(ridx.at[s, pl.ds(0, nch0)],
                                ridx_v.at[pl.ds(0, nch0)])
                pltpu.sync_copy(cidx.at[s, pl.ds(0, nch0)],
                                cidx_v.at[pl.ds(0, nch0)])

        @pl.when(c == 1)
        def _():
            if nch1:
                pltpu.sync_copy(ridx.at[s, pl.ds(nch0, nch1)],
                                ridx_v.at[pl.ds(0, nch1)])
                pltpu.sync_copy(cidx.at[s, pl.ds(nch0, nch1)],
                                cidx_v.at[pl.ds(0, nch1)])

        plsc.subcore_barrier()
        _run_pass(c, table_a, ridx_v, cidx_v, bufs, acc, sem_g, sem_s,
                  zrows, nch0, nch1)
        plsc.subcore_barrier()
        pltpu.sync_copy(acc.at[pl.ds(base, orow)],
                        out.at[0, c, pl.ds(base, orow)])
        zero_own_slice()
        plsc.subcore_barrier()
        _run_pass(c, table_b, ridx_v, cidx_v, bufs, acc, sem_g, sem_s,
                  zrows, nch0, nch1)
        plsc.subcore_barrier()
        pltpu.sync_copy(acc.at[pl.ds(base, orow)],
                        out.at[1, c, pl.ds(base, orow)])

    return qk


@functools.lru_cache(maxsize=None)
def _make_deg(n, nch0, nch1):
    npad = _npad(n)
    orow = npad // _NS
    zfull, zrem = orow // _CH, orow % _CH
    nchv = max(nch0, nch1)
    mesh = plsc.VectorSubcoreMesh(core_axis_name="c", subcore_axis_name="s",
                                  num_cores=_NC, num_subcores=_NS)

    @functools.partial(
        pl.kernel,
        out_type=jax.ShapeDtypeStruct((_NC, npad, _DEGW), jnp.float32),
        mesh=mesh,
        compiler_params=pltpu.CompilerParams(use_tc_tiling_on_sc=False),
        scratch_types=[
            pltpu.VMEM((nchv, _CH), jnp.int32),
            pltpu.VMEM((_CH, _DEGW), jnp.float32),
            pltpu.VMEM((_CH, _DEGW), jnp.float32),
            pltpu.VMEM_SHARED((npad, _DEGW), jnp.float32),
            pltpu.SemaphoreType.DMA,
        ],
    )
    def dk(ridx, orows, zrows, out, ridx_v, obuf, zbuf, acc, sem):
        c = lax.axis_index("c")
        s = lax.axis_index("s")
        pltpu.sync_copy(zrows, zbuf)
        pltpu.sync_copy(orows, obuf)
        base = s * orow
        for j in range(zfull):
            pltpu.sync_copy(zbuf, acc.at[pl.ds(base + j * _CH, _CH)])
        if zrem:
            pltpu.sync_copy(zbuf.at[pl.ds(0, zrem)],
                            acc.at[pl.ds(base + zfull * _CH, zrem)])

        @pl.when(c == 0)
        def _():
            if nch0:
                pltpu.sync_copy(ridx.at[s, pl.ds(0, nch0)],
                                ridx_v.at[pl.ds(0, nch0)])

        @pl.when(c == 1)
        def _():
            if nch1:
                pltpu.sync_copy(ridx.at[s, pl.ds(nch0, nch1)],
                                ridx_v.at[pl.ds(0, nch1)])

        plsc.subcore_barrier()
        nch_c = jnp.where(c == 0, nch0, nch1)

        def chunk(j, carry):
            @pl.when(j >= _NB)
            def _():
                pltpu.make_async_copy(orows, obuf, sem).wait()
            pltpu.async_copy(obuf, acc.at[ridx_v.at[j]], sem, add=True)
            return carry

        lax.fori_loop(0, nch_c, chunk, 0)

        def drain(j, carry):
            pltpu.make_async_copy(orows, obuf, sem).wait()
            return carry

        lax.fori_loop(0, jnp.minimum(nch_c, _NB), drain, 0)
        plsc.subcore_barrier()
        pltpu.sync_copy(acc.at[pl.ds(s * orow, orow)],
                        out.at[c, pl.ds(s * orow, orow)])

    return dk


def _dis_of(dg_ref, n):
    dg = dg_ref[...]
    deg = dg[0, :n] + dg[1, :n]
    dis = jnp.where(deg > 0, lax.rsqrt(jnp.maximum(deg, 1e-12)), 0.0)
    return dis[:, 0:1]


def _psum(p_ref, n):
    p = p_ref[...]
    return p[0, :n] + p[1, :n]


def _tc1_body(x_ref, w_ref, b_ref, dg_ref, sa1_ref, sa2_ref, st_ref):
    dh = st_ref.shape[1]
    dis = _dis_of(dg_ref, x_ref.shape[0])
    a = jnp.dot(x_ref[...], w_ref[...], preferred_element_type=jnp.float32)
    sa1_ref[...] = a[:, dh:2 * dh] * dis
    sa2_ref[...] = a[:, 2 * dh:] * dis
    st_ref[...] = a[:, :dh] - a[:, 2 * dh:] + b_ref[...]


def _tc3_body(g12_ref, dg_ref, st_ref, in2_ref, p1_ref):
    n = st_ref.shape[0]
    dis = _dis_of(dg_ref, n)
    g12 = g12_ref[...]
    g1a = g12[0, 0, :n] + g12[0, 1, :n]
    g1b = g12[1, 0, :n] + g12[1, 1, :n]
    in2_ref[...] = g1b * (dis * dis)
    p1_ref[...] = st_ref[...] - g1a * dis


def _tc5_body(g2_ref, dg_ref, p1_ref, g_ref, be_ref, h_ref, hs_ref):
    n = p1_ref.shape[0]
    dis = _dis_of(dg_ref, n)
    g2s = _psum(g2_ref, n)
    h1 = p1_ref[...] + 2.0 * dis * g2s
    m = jnp.mean(h1, axis=0, keepdims=True)
    v = jnp.mean((h1 - m) ** 2, axis=0, keepdims=True)
    hb = (h1 - m) * lax.rsqrt(v + 1e-5) * g_ref[...] + be_ref[...]
    h = jnp.where(hb >= 0, hb, 0.01 * hb)
    h_ref[...] = h
    hs_ref[...] = h * dis


def _tc7_body(q1_ref, dg_ref, h_ref, w2_ref, b2_ref, in3_ref, acc_ref):
    n = h_ref.shape[0]
    dis = _dis_of(dg_ref, n)
    q1s = _psum(q1_ref, n)
    tx1 = -dis * q1s
    in3_ref[...] = dis * tx1
    w2 = w2_ref[...]
    acc_ref[...] = (jnp.dot(h_ref[...], w2[0], preferred_element_type=jnp.float32)
                    + jnp.dot(tx1, w2[1], preferred_element_type=jnp.float32)
                    + b2_ref[...])


def _tc9_body(q2_ref, dg_ref, acc_ref, h_ref, w2_ref, wf_ref, bf_ref, out_ref):
    n = h_ref.shape[0]
    dis = _dis_of(dg_ref, n)
    q2s = _psum(q2_ref, n)
    tx2 = -2.0 * dis * q2s - h_ref[...]
    o = acc_ref[...] + jnp.dot(tx2, w2_ref[...][2],
                               preferred_element_type=jnp.float32)
    out_ref[...] = jnp.dot(o, wf_ref[...],
                           preferred_element_type=jnp.float32) + bf_ref[...]


def kernel(x, edge_index, W1, b1, g1, be1, W2, b2, Wf, bf):
    n, d_in = x.shape
    kc, _, dh = W1.shape
    assert kc == 3, "kernel specialized for K=3 Chebyshev order"
    e = edge_index.shape[1]
    nclass = Wf.shape[1]

    grp = 2 * _NB
    per_s = -(-e // _NS)
    ncht = -(-per_s // _CH)
    f0 = 0.65
    nch0 = min(-(-ncht // grp) * grp,
               max(grp, int(round(ncht * f0 / grp)) * grp))
    nch1 = max(0, -(-(ncht - nch0) // 2) * 2)
    ncht = nch0 + nch1
    epad = _NS * ncht * _CH
    padn = epad - e

    row = edge_index[0]
    col = edge_index[1]
    ridx_g = jnp.concatenate(
        [row, jnp.zeros((padn,), jnp.int32)]).reshape(_NS, ncht, _CH)
    ridx_d = jnp.concatenate(
        [row, jnp.full((padn,), n, jnp.int32)]).reshape(_NS, ncht, _CH)
    cidx = jnp.concatenate(
        [col, jnp.full((padn,), n, jnp.int32)]).reshape(_NS, ncht, _CH)

    zq64 = jnp.zeros((_CH, dh), jnp.float32)
    zdeg = jnp.zeros((_CH, _DEGW), jnp.float32)
    odeg = jnp.ones((_CH, _DEGW), jnp.float32)

    w1r = jnp.transpose(W1, (1, 0, 2)).reshape(d_in, kc * dh)

    degp = _make_deg(n, nch0, nch1)(ridx_d, odeg, zdeg)

    sa1, sa2, stash = pl.pallas_call(
        _tc1_body,
        out_shape=(jax.ShapeDtypeStruct((n, dh), jnp.float32),
                   jax.ShapeDtypeStruct((n, dh), jnp.float32),
                   jax.ShapeDtypeStruct((n, dh), jnp.float32)),
    )(x, w1r, b1.reshape(1, dh), degp)

    g12p = _make_q2(n, dh, nch0, nch1)(sa1, sa2, ridx_g, cidx, zq64)

    in2, p1 = pl.pallas_call(
        _tc3_body,
        out_shape=(jax.ShapeDtypeStruct((n, dh), jnp.float32),
                   jax.ShapeDtypeStruct((n, dh), jnp.float32)),
    )(g12p, degp, stash)

    g2p = _make_q(n, dh, nch0, nch1)(in2, ridx_g, cidx, zq64)

    h, hs = pl.pallas_call(
        _tc5_body,
        out_shape=(jax.ShapeDtypeStruct((n, dh), jnp.float32),
                   jax.ShapeDtypeStruct((n, dh), jnp.float32)),
    )(g2p, degp, p1, g1.reshape(1, dh), be1.reshape(1, dh))

    q1p = _make_q(n, dh, nch0, nch1)(hs, ridx_g, cidx, zq64)

    in3, acc2 = pl.pallas_call(
        _tc7_body,
        out_shape=(jax.ShapeDtypeStruct((n, dh), jnp.float32),
                   jax.ShapeDtypeStruct((n, dh), jnp.float32)),
    )(q1p, degp, h, W2, b2.reshape(1, dh))

    q2p = _make_q(n, dh, nch0, nch1)(in3, ridx_g, cidx, zq64)

    out = pl.pallas_call(
        _tc9_body,
        out_shape=jax.ShapeDtypeStruct((n, nclass), jnp.float32),
    )(q2p, degp, acc2, h, W2, Wf, bf.reshape(1, nclass))

    return out


# speedup vs baseline: 1.3714x; 1.3714x over previous
import functools

import jax
import jax.numpy as jnp
from jax import lax
from jax.experimental import pallas as pl
from jax.experimental.pallas import tpu as pltpu
from jax.experimental.pallas import tpu_sc as plsc

_NC = 2
_NS = 16
_NW = _NC * _NS
_CH = 128
_NB = 3
_DEGW = 8


def _npad(n):
    return ((n + _NS + _NS - 1) // _NS) * _NS


def _run_pass(c, table, ridx_v, cidx_v, bufs, acc, sem_g, sem_s, zrows,
              nch0, nch1):
    @pl.when(c == 0)
    def _():
        def pair(g2, carry):
            bbase = g2 * 2 * _NB
            for p in range(2):
                @pl.when(g2 > 0)
                def _():
                    for i in range(_NB):
                        pltpu.make_async_copy(zrows, bufs[p][i],
                                              sem_s[p]).wait()
                for i in range(_NB):
                    j = bbase + p * _NB + i
                    pltpu.async_copy(table.at[ridx_v.at[j]], bufs[p][i],
                                     sem_g[p])
                for i in range(_NB):
                    j = bbase + p * _NB + i
                    pltpu.make_async_copy(zrows, bufs[p][i], sem_g[p]).wait()
                    pltpu.async_copy(bufs[p][i], acc.at[cidx_v.at[j]],
                                     sem_s[p], add=True)
            return carry

        lax.fori_loop(0, nch0 // (2 * _NB), pair, 0)
        if nch0:
            for p in range(2):
                for i in range(_NB):
                    pltpu.make_async_copy(zrows, bufs[p][i], sem_s[p]).wait()

    @pl.when(c == 1)
    def _():
        if nch1:
            def sch(j, carry):
                pltpu.async_copy(table.at[ridx_v.at[j]], bufs[0][0],
                                 sem_g[1]).wait()
                pltpu.sync_copy(bufs[0][0], acc.at[cidx_v.at[j]], add=True)
                return carry

            lax.fori_loop(0, nch1, sch, 0)


@functools.lru_cache(maxsize=None)
def _make_q(n, d, nch0, nch1):
    npad = _npad(n)
    orow = npad // _NS
    zfull, zrem = orow // _CH, orow % _CH
    nchv = max(nch0, nch1)
    mesh = plsc.VectorSubcoreMesh(core_axis_name="c", subcore_axis_name="s",
                                  num_cores=_NC, num_subcores=_NS)

    @functools.partial(
        pl.kernel,
        out_type=jax.ShapeDtypeStruct((_NC, npad, d), jnp.float32),
        mesh=mesh,
        compiler_params=pltpu.CompilerParams(use_tc_tiling_on_sc=False),
        scratch_types=[
            pltpu.VMEM((nchv, _CH), jnp.int32),
            pltpu.VMEM((nchv, _CH), jnp.int32),
            [[pltpu.VMEM((_CH, d), jnp.float32) for _ in range(_NB)]
             for _ in range(2)],
            pltpu.VMEM_SHARED((npad, d), jnp.float32),
            [pltpu.SemaphoreType.DMA for _ in range(2)],
            [pltpu.SemaphoreType.DMA for _ in range(2)],
        ],
    )
    def qk(table, ridx, cidx, zrows, out, ridx_v, cidx_v, bufs, acc,
           sem_g, sem_s):
        c = lax.axis_index("c")
        s = lax.axis_index("s")
        base = s * orow
        for j in range(zfull):
            pltpu.sync_copy(zrows, acc.at[pl.ds(base + j * _CH, _CH)])
        if zrem:
            pltpu.sync_copy(zrows.at[pl.ds(0, zrem)],
                            acc.at[pl.ds(base + zfull * _CH, zrem)])

        @pl.when(c == 0)
        def _():
            if nch0:
                pltpu.sync_copy(ridx.at[s, pl.ds(0, nch0)],
                                ridx_v.at[pl.ds(0, nch0)])
                pltpu.sync_copy(cidx.at[s, pl.ds(0, nch0)],
                                cidx_v.at[pl.ds(0, nch0)])

        @pl.when(c == 1)
        def _():
            if nch1:
                pltpu.sync_copy(ridx.at[s, pl.ds(nch0, nch1)],
                                ridx_v.at[pl.ds(0, nch1)])
                pltpu.sync_copy(cidx.at[s, pl.ds(nch0, nch1)],
                                cidx_v.at[pl.ds(0, nch1)])

        plsc.subcore_barrier()
        _run_pass(c, table, ridx_v, cidx_v, bufs, acc, sem_g, sem_s, zrows,
                  nch0, nch1)
        plsc.subcore_barrier()
        pltpu.sync_copy(acc.at[pl.ds(s * orow, orow)],
                        out.at[c, pl.ds(s * orow, orow)])

    return qk


@functools.lru_cache(maxsize=None)
def _make_q2(n, d, nch0, nch1):
    npad = _npad(n)
    orow = npad // _NS
    zfull, zrem = orow // _CH, orow % _CH
    nchv = max(nch0, nch1)
    mesh = plsc.VectorSubcoreMesh(core_axis_name="c", subcore_axis_name="s",
                                  num_cores=_NC, num_subcores=_NS)

    @functools.partial(
        pl.kernel,
        out_type=jax.ShapeDtypeStruct((2, _NC, npad, d), jnp.float32),
        mesh=mesh,
        compiler_params=pltpu.CompilerParams(use_tc_tiling_on_sc=False),
        scratch_types=[
            pltpu.VMEM((nchv, _CH), jnp.int32),
            pltpu.VMEM((nchv, _CH), jnp.int32),
            [[pltpu.VMEM((_CH, d), jnp.float32) for _ in range(_NB)]
             for _ in range(2)],
            pltpu.VMEM_SHARED((npad, d), jnp.float32),
            [pltpu.SemaphoreType.DMA for _ in range(2)],
            [pltpu.SemaphoreType.DMA for _ in range(2)],
        ],
    )
    def qk(table_a, table_b, ridx, cidx, zrows, out, ridx_v, cidx_v, bufs,
           acc, sem_g, sem_s):
        c = lax.axis_index("c")
        s = lax.axis_index("s")
        base = s * orow

        def zero_own_slice():
            for j in range(zfull):
                pltpu.sync_copy(zrows, acc.at[pl.ds(base + j * _CH, _CH)])
            if zrem:
                pltpu.sync_copy(zrows.at[pl.ds(0, zrem)],
                                acc.at[pl.ds(base + zfull * _CH, zrem)])

        zero_own_slice()

        @pl.when(c == 0)
        def _():
            if nch0:
                pltpu.sync_copy(ridx.at[s, pl.ds(0, nch0)],
                                ridx_v.at[pl.ds(0, nch0)])
                pltpu.sync_copy(cidx.at[s, pl.ds(0, nch0)],
                                cidx_v.at[pl.ds(0, nch0)])

        @pl.when(c == 1)
        def _():
            if nch1:
                pltpu.sync_copy(ridx.at[s, pl.ds(nch0, nch1)],
                                ridx_v.at[pl.ds(0, nch1)])
                pltpu.sync_copy(cidx.at[s, pl.ds(nch0, nch1)],
                                cidx_v.at[pl.ds(0, nch1)])

        plsc.subcore_barrier()
        _run_pass(c, table_a, ridx_v, cidx_v, bufs, acc, sem_g, sem_s,
                  zrows, nch0, nch1)
        plsc.subcore_barrier()
        pltpu.sync_copy(acc.at[pl.ds(base, orow)],
                        out.at[0, c, pl.ds(base, orow)])
        zero_own_slice()
        plsc.subcore_barrier()
        _run_pass(c, table_b, ridx_v, cidx_v, bufs, acc, sem_g, sem_s,
                  zrows, nch0, nch1)
        plsc.subcore_barrier()
        pltpu.sync_copy(acc.at[pl.ds(base, orow)],
                        out.at[1, c, pl.ds(base, orow)])

    return qk


@functools.lru_cache(maxsize=None)
def _make_deg(n, nch0, nch1):
    npad = _npad(n)
    orow = npad // _NS
    zfull, zrem = orow // _CH, orow % _CH
    nchv = max(nch0, nch1)
    mesh = plsc.VectorSubcoreMesh(core_axis_name="c", subcore_axis_name="s",
                                  num_cores=_NC, num_subcores=_NS)

    @functools.partial(
        pl.kernel,
        out_type=jax.ShapeDtypeStruct((_NC, npad, _DEGW), jnp.float32),
        mesh=mesh,
        compiler_params=pltpu.CompilerParams(use_tc_tiling_on_sc=False),
        scratch_types=[
            pltpu.VMEM((nchv, _CH), jnp.int32),
            pltpu.VMEM((_CH, _DEGW), jnp.float32),
            pltpu.VMEM((_CH, _DEGW), jnp.float32),
            pltpu.VMEM_SHARED((npad, _DEGW), jnp.float32),
            pltpu.SemaphoreType.DMA,
        ],
    )
    def dk(ridx, orows, zrows, out, ridx_v, obuf, zbuf, acc, sem):
        c = lax.axis_index("c")
        s = lax.axis_index("s")
        pltpu.sync_copy(zrows, zbuf)
        pltpu.sync_copy(orows, obuf)
        base = s * orow
        for j in range(zfull):
            pltpu.sync_copy(zbuf, acc.at[pl.ds(base + j * _CH, _CH)])
        if zrem:
            pltpu.sync_copy(zbuf.at[pl.ds(0, zrem)],
                            acc.at[pl.ds(base + zfull * _CH, zrem)])

        @pl.when(c == 0)
        def _():
            if nch0:
                pltpu.sync_copy(ridx.at[s, pl.ds(0, nch0)],
                                ridx_v.at[pl.ds(0, nch0)])

        @pl.when(c == 1)
        def _():
            if nch1:
                pltpu.sync_copy(ridx.at[s, pl.ds(nch0, nch1)],
                                ridx_v.at[pl.ds(0, nch1)])

        plsc.subcore_barrier()
        nch_c = jnp.where(c == 0, nch0, nch1)

        def chunk(j, carry):
            @pl.when(j >= _NB)
            def _():
                pltpu.make_async_copy(orows, obuf, sem).wait()
            pltpu.async_copy(obuf, acc.at[ridx_v.at[j]], sem, add=True)
            return carry

        lax.fori_loop(0, nch_c, chunk, 0)

        def drain(j, carry):
            pltpu.make_async_copy(orows, obuf, sem).wait()
            return carry

        lax.fori_loop(0, jnp.minimum(nch_c, _NB), drain, 0)
        plsc.subcore_barrier()
        pltpu.sync_copy(acc.at[pl.ds(s * orow, orow)],
                        out.at[c, pl.ds(s * orow, orow)])

    return dk


def _dis_of(dg_ref, n):
    dg = dg_ref[...]
    deg = dg[0, :n] + dg[1, :n]
    dis = jnp.where(deg > 0, lax.rsqrt(jnp.maximum(deg, 1e-12)), 0.0)
    return dis[:, 0:1]


def _psum(p_ref, n):
    p = p_ref[...]
    return p[0, :n] + p[1, :n]


def _tc1_body(x_ref, w_ref, b_ref, dg_ref, sa1_ref, sa2_ref, st_ref):
    dh = st_ref.shape[1]
    dis = _dis_of(dg_ref, x_ref.shape[0])
    a = jnp.dot(x_ref[...], w_ref[...], preferred_element_type=jnp.float32)
    sa1_ref[...] = a[:, dh:2 * dh] * dis
    sa2_ref[...] = a[:, 2 * dh:] * dis
    st_ref[...] = a[:, :dh] - a[:, 2 * dh:] + b_ref[...]


def _tc3_body(g12_ref, dg_ref, st_ref, in2_ref, p1_ref):
    n = st_ref.shape[0]
    dis = _dis_of(dg_ref, n)
    g12 = g12_ref[...]
    g1a = g12[0, 0, :n] + g12[0, 1, :n]
    g1b = g12[1, 0, :n] + g12[1, 1, :n]
    in2_ref[...] = g1b * (dis * dis)
    p1_ref[...] = st_ref[...] - g1a * dis


def _tc5_body(g2_ref, dg_ref, p1_ref, g_ref, be_ref, h_ref, hs_ref):
    n = p1_ref.shape[0]
    dis = _dis_of(dg_ref, n)
    g2s = _psum(g2_ref, n)
    h1 = p1_ref[...] + 2.0 * dis * g2s
    m = jnp.mean(h1, axis=0, keepdims=True)
    v = jnp.mean((h1 - m) ** 2, axis=0, keepdims=True)
    hb = (h1 - m) * lax.rsqrt(v + 1e-5) * g_ref[...] + be_ref[...]
    h = jnp.where(hb >= 0, hb, 0.01 * hb)
    h_ref[...] = h
    hs_ref[...] = h * dis


def _tc7_body(q1_ref, dg_ref, h_ref, w2_ref, b2_ref, in3_ref, acc_ref):
    n = h_ref.shape[0]
    dis = _dis_of(dg_ref, n)
    q1s = _psum(q1_ref, n)
    tx1 = -dis * q1s
    in3_ref[...] = dis * tx1
    w2 = w2_ref[...]
    acc_ref[...] = (jnp.dot(h_ref[...], w2[0], preferred_element_type=jnp.float32)
                    + jnp.dot(tx1, w2[1], preferred_element_type=jnp.float32)
                    + b2_ref[...])


def _tc9_body(q2_ref, dg_ref, acc_ref, h_ref, w2_ref, wf_ref, bf_ref, out_ref):
    n = h_ref.shape[0]
    dis = _dis_of(dg_ref, n)
    q2s = _psum(q2_ref, n)
    tx2 = -2.0 * dis * q2s - h_ref[...]
    o = acc_ref[...] + jnp.dot(tx2, w2_ref[...][2],
                               preferred_element_type=jnp.float32)
    out_ref[...] = jnp.dot(o, wf_ref[...],
                           preferred_element_type=jnp.float32) + bf_ref[...]


def kernel(x, edge_index, W1, b1, g1, be1, W2, b2, Wf, bf):
    n, d_in = x.shape
    kc, _, dh = W1.shape
    assert kc == 3, "kernel specialized for K=3 Chebyshev order"
    e = edge_index.shape[1]
    nclass = Wf.shape[1]

    grp = 2 * _NB
    per_s = -(-e // _NS)
    ncht = -(-per_s // _CH)
    f0 = 0.765
    nch0 = min(-(-ncht // grp) * grp,
               max(grp, int(round(ncht * f0 / grp)) * grp))
    nch1 = max(0, ncht - nch0)
    ncht = nch0 + nch1
    epad = _NS * ncht * _CH
    padn = epad - e

    row = edge_index[0]
    col = edge_index[1]
    ridx_g = jnp.concatenate(
        [row, jnp.zeros((padn,), jnp.int32)]).reshape(_NS, ncht, _CH)
    ridx_d = jnp.concatenate(
        [row, jnp.full((padn,), n, jnp.int32)]).reshape(_NS, ncht, _CH)
    cidx = jnp.concatenate(
        [col, jnp.full((padn,), n, jnp.int32)]).reshape(_NS, ncht, _CH)

    zq64 = jnp.zeros((_CH, dh), jnp.float32)
    zdeg = jnp.zeros((_CH, _DEGW), jnp.float32)
    odeg = jnp.ones((_CH, _DEGW), jnp.float32)

    w1r = jnp.transpose(W1, (1, 0, 2)).reshape(d_in, kc * dh)

    degp = _make_deg(n, nch0, nch1)(ridx_d, odeg, zdeg)

    sa1, sa2, stash = pl.pallas_call(
        _tc1_body,
        out_shape=(jax.ShapeDtypeStruct((n, dh), jnp.float32),
                   jax.ShapeDtypeStruct((n, dh), jnp.float32),
                   jax.ShapeDtypeStruct((n, dh), jnp.float32)),
    )(x, w1r, b1.reshape(1, dh), degp)

    g12p = _make_q2(n, dh, nch0, nch1)(sa1, sa2, ridx_g, cidx, zq64)

    in2, p1 = pl.pallas_call(
        _tc3_body,
        out_shape=(jax.ShapeDtypeStruct((n, dh), jnp.float32),
                   jax.ShapeDtypeStruct((n, dh), jnp.float32)),
    )(g12p, degp, stash)

    g2p = _make_q(n, dh, nch0, nch1)(in2, ridx_g, cidx, zq64)

    h, hs = pl.pallas_call(
        _tc5_body,
        out_shape=(jax.ShapeDtypeStruct((n, dh), jnp.float32),
                   jax.ShapeDtypeStruct((n, dh), jnp.float32)),
    )(g2p, degp, p1, g1.reshape(1, dh), be1.reshape(1, dh))

    q1p = _make_q(n, dh, nch0, nch1)(hs, ridx_g, cidx, zq64)

    in3, acc2 = pl.pallas_call(
        _tc7_body,
        out_shape=(jax.ShapeDtypeStruct((n, dh), jnp.float32),
                   jax.ShapeDtypeStruct((n, dh), jnp.float32)),
    )(q1p, degp, h, W2, b2.reshape(1, dh))

    q2p = _make_q(n, dh, nch0, nch1)(in3, ridx_g, cidx, zq64)

    out = pl.pallas_call(
        _tc9_body,
        out_shape=jax.ShapeDtypeStruct((n, nclass), jnp.float32),
    )(q2p, degp, acc2, h, W2, Wf, bf.reshape(1, nclass))

    return out
